# baseline (speedup 1.0000x reference)
import functools

import jax
import jax.numpy as jnp
from jax import lax
from jax.experimental import pallas as pl
from jax.experimental.pallas import tpu as pltpu
from jax.experimental.pallas import tpu_sc as plsc

N_DOF = 261121
NNZ = 1827847
NW = 32
C = 5728
CB = 5760
T = 10
NBUF = 4
MU = 12
N_PAD = 262144
U_SL = N_PAD // 16
Z = 4096

S_CHUNKS = NW * T
FULL_CHUNKS = S_CHUNKS - 1
SP_OFF = FULL_CHUNKS * C
SP_CNT = NNZ - SP_OFF
SP_G = SP_CNT // 16
SP_R = SP_CNT % 16

_mesh = plsc.VectorSubcoreMesh(core_axis_name="c", subcore_axis_name="s")


@functools.partial(
    pl.kernel,
    out_type=jax.ShapeDtypeStruct((2 * N_PAD,), jnp.float32),
    mesh=_mesh,
    scratch_types=[
        pltpu.VMEM_SHARED((N_PAD,), jnp.float32),
        pltpu.VMEM_SHARED((N_PAD,), jnp.float32),
        *([pltpu.VMEM((CB,), jnp.int32)] * NBUF),
        *([pltpu.VMEM((CB,), jnp.int32)] * NBUF),
        *([pltpu.VMEM((CB,), jnp.float32)] * NBUF),
        *([pltpu.VMEM((CB,), jnp.float32)] * NBUF),
        pltpu.VMEM((Z,), jnp.float32),
        *([pltpu.SemaphoreType.DMA] * NBUF),
        *([pltpu.SemaphoreType.DMA] * NBUF),
        *([pltpu.SemaphoreType.DMA] * NBUF),
    ],
)
def _spmv_sc(u_hbm, rows_hbm, cols_hbm, vals_hbm, out_hbm,
             u_s, acc_s,
             rows_v0, rows_v1, rows_v2, rows_v3,
             cols_v0, cols_v1, cols_v2, cols_v3,
             vals_v0, vals_v1, vals_v2, vals_v3,
             g_v0, g_v1, g_v2, g_v3, z_v,
             si0, si1, si2, si3, sg0, sg1, sg2, sg3, ss0, ss1, ss2, ss3):
    rows_b = (rows_v0, rows_v1, rows_v2, rows_v3)
    cols_b = (cols_v0, cols_v1, cols_v2, cols_v3)
    vals_b = (vals_v0, vals_v1, vals_v2, vals_v3)
    g_b = (g_v0, g_v1, g_v2, g_v3)
    sem_in = (si0, si1, si2, si3)
    sem_g = (sg0, sg1, sg2, sg3)
    sem_sc = (ss0, ss1, ss2, ss3)

    cid = lax.axis_index("c")
    sid = lax.axis_index("s")
    wid = sid * 2 + cid

    def start_in(gidx, b):
        @pl.when(gidx < FULL_CHUNKS)
        def _():
            off = gidx * C
            pltpu.async_copy(rows_hbm.at[pl.ds(off, C)],
                             rows_b[b].at[pl.ds(0, C)], sem_in[b])
            pltpu.async_copy(cols_hbm.at[pl.ds(off, C)],
                             cols_b[b].at[pl.ds(0, C)], sem_in[b])
            pltpu.async_copy(vals_hbm.at[pl.ds(off, C)],
                             vals_b[b].at[pl.ds(0, C)], sem_in[b])

        @pl.when(gidx == FULL_CHUNKS)
        def _():
            pltpu.async_copy(rows_hbm.at[pl.ds(SP_OFF, SP_CNT)],
                             rows_b[b].at[pl.ds(0, SP_CNT)], sem_in[b])
            pltpu.async_copy(cols_hbm.at[pl.ds(SP_OFF, SP_CNT)],
                             cols_b[b].at[pl.ds(0, SP_CNT)], sem_in[b])
            pltpu.async_copy(vals_hbm.at[pl.ds(SP_OFF, SP_CNT)],
                             vals_b[b].at[pl.ds(0, SP_CNT)], sem_in[b])

    def wait_in(gidx, b):
        @pl.when(gidx < FULL_CHUNKS)
        def _():
            pltpu.make_async_copy(rows_hbm.at[pl.ds(0, C)],
                                  rows_b[b].at[pl.ds(0, C)], sem_in[b]).wait()
            pltpu.make_async_copy(cols_hbm.at[pl.ds(0, C)],
                                  cols_b[b].at[pl.ds(0, C)], sem_in[b]).wait()
            pltpu.make_async_copy(vals_hbm.at[pl.ds(0, C)],
                                  vals_b[b].at[pl.ds(0, C)], sem_in[b]).wait()

        @pl.when(gidx == FULL_CHUNKS)
        def _():
            pltpu.make_async_copy(rows_hbm.at[pl.ds(0, SP_CNT)],
                                  rows_b[b].at[pl.ds(0, SP_CNT)], sem_in[b]).wait()
            pltpu.make_async_copy(cols_hbm.at[pl.ds(0, SP_CNT)],
                                  cols_b[b].at[pl.ds(0, SP_CNT)], sem_in[b]).wait()
            pltpu.make_async_copy(vals_hbm.at[pl.ds(0, SP_CNT)],
                                  vals_b[b].at[pl.ds(0, SP_CNT)], sem_in[b]).wait()

    H = CB // 2

    def _u_src(b):
        return u_hbm if b % 2 else u_s

    def start_gather(b):
        pltpu.async_copy(_u_src(b).at[cols_b[b].at[pl.ds(0, H)]],
                         g_b[b].at[pl.ds(0, H)], sem_g[b])
        pltpu.async_copy(_u_src(b).at[cols_b[b].at[pl.ds(H, H)]],
                         g_b[b].at[pl.ds(H, H)], sem_g[b])

    def wait_gather(b):
        pltpu.make_async_copy(_u_src(b).at[cols_b[b].at[pl.ds(0, H)]],
                              g_b[b].at[pl.ds(0, H)], sem_g[b]).wait()
        pltpu.make_async_copy(_u_src(b).at[cols_b[b].at[pl.ds(H, H)]],
                              g_b[b].at[pl.ds(H, H)], sem_g[b]).wait()

    def start_sc(b):
        pltpu.async_copy(g_b[b], acc_s.at[rows_b[b]], sem_sc[b], add=True)

    def wait_sc(b):
        pltpu.make_async_copy(g_b[b], acc_s.at[rows_b[b]], sem_sc[b]).wait()

    def gi(t):
        return t * NW + wid

    start_in(gi(0), 0)
    start_in(gi(1), 1)
    pltpu.async_copy(u_hbm.at[pl.ds(sid * U_SL, U_SL)],
                     u_s.at[pl.ds(sid * U_SL, U_SL)], sem_g[0])

    def zset(i, _):
        z_v[pl.ds(i * 16, 16)] = jnp.zeros((16,), jnp.float32)
        return 0
    lax.fori_loop(0, Z // 16, zset, 0)

    def zcpy(j, _):
        pltpu.sync_copy(z_v, acc_s.at[pl.ds(sid * U_SL + j * Z, Z)])
        return 0
    lax.fori_loop(0, U_SL // Z, zcpy, 0)

    widv = jnp.full((16,), wid, jnp.int32)
    for b in range(NBUF):
        for i in range(C // 16, CB // 16):
            s = pl.ds(i * 16, 16)
            rows_b[b][s] = widv
            cols_b[b][s] = widv
            vals_b[b][s] = jnp.zeros((16,), jnp.float32)

    pltpu.make_async_copy(u_hbm.at[pl.ds(sid * U_SL, U_SL)],
                          u_s.at[pl.ds(sid * U_SL, U_SL)], sem_g[0]).wait()
    plsc.subcore_barrier()

    wait_in(gi(0), 0)
    start_gather(0)

    def consume(t, k):
        b = k
        wait_gather(b)
        gb, vb = g_b[b], vals_b[b]

        @pl.when(gi(t) == FULL_CHUNKS)
        def _():
            keep = jax.lax.iota(jnp.int32, 16) < SP_R
            sb = pl.ds(SP_G * 16, 16)
            vb[sb] = jnp.where(keep, vb[sb], 0.0)

            def zdup(i, _):
                vb[pl.ds(i * 16, 16)] = jnp.zeros((16,), jnp.float32)
                return 0
            lax.fori_loop(SP_G + 1, C // 16, zdup, 0)

        def mul(i, _):
            for r in range(MU):
                s = pl.ds(i * (16 * MU) + r * 16, 16)
                gb[s] = gb[s] * vb[s]
            return 0
        lax.fori_loop(0, CB // (16 * MU), mul, 0)

        start_sc(b)

    def body4(j, _):
        for k in range(NBUF):
            t = j * NBUF + k

            @pl.when(t >= 2)
            def _():
                wait_sc((k - 2) % NBUF)

            start_in(gi(t + 2), (k + 2) % NBUF)
            wait_in(gi(t + 1), (k + 1) % NBUF)
            start_gather((k + 1) % NBUF)
            consume(t, k)
        return 0
    lax.fori_loop(0, (T - 2) // NBUF, body4, 0)

    kA, kB = (T - 2) % NBUF, (T - 1) % NBUF
    wait_sc((kA - 2) % NBUF)
    wait_in(gi(T - 1), kB)
    start_gather(kB)
    consume(T - 2, kA)
    wait_sc((kB - 2) % NBUF)
    consume(T - 1, kB)
    wait_sc(kA)
    wait_sc(kB)

    plsc.subcore_barrier()
    pltpu.sync_copy(acc_s.at[pl.ds(sid * U_SL, U_SL)],
                    out_hbm.at[pl.ds(cid * N_PAD + sid * U_SL, U_SL)])


def _add_body(p_ref, o_ref):
    o_ref[...] = p_ref[pl.ds(0, N_PAD)] + p_ref[pl.ds(N_PAD, N_PAD)]


def kernel(u, A_rows, A_cols, A_vals):
    u_p = jnp.concatenate(
        [u, jnp.zeros((N_PAD - N_DOF, 1), jnp.float32)]).reshape(N_PAD)
    partials = _spmv_sc(u_p, A_rows.astype(jnp.int32),
                        A_cols.astype(jnp.int32), A_vals)

    summed = pl.pallas_call(
        _add_body,
        out_shape=jax.ShapeDtypeStruct((N_PAD,), jnp.float32),
    )(partials)
    return summed[:N_DOF, None]


# speedup vs baseline: 1.2422x; 1.2422x over previous
import functools

import jax
import jax.numpy as jnp
from jax import lax
from jax.experimental import pallas as pl
from jax.experimental.pallas import tpu as pltpu
from jax.experimental.pallas import tpu_sc as plsc

N_DOF = 261121
NNZ = 1827847
NW = 32
C = 5728
CB = 5760
T = 10
NBUF = 4
MU = 12
N_PAD = 262144
U_SL = N_PAD // 16
Z = 4096

S_CHUNKS = NW * T
FULL_CHUNKS = S_CHUNKS - 1
SP_OFF = FULL_CHUNKS * C
SP_CNT = NNZ - SP_OFF
SP_G = SP_CNT // 16
SP_R = SP_CNT % 16

_mesh = plsc.VectorSubcoreMesh(core_axis_name="c", subcore_axis_name="s")


@functools.partial(
    pl.kernel,
    out_type=jax.ShapeDtypeStruct((2 * N_PAD,), jnp.float32),
    mesh=_mesh,
    scratch_types=[
        pltpu.VMEM_SHARED((N_PAD,), jnp.float32),
        pltpu.VMEM_SHARED((N_PAD,), jnp.float32),
        *([pltpu.VMEM((CB,), jnp.int32)] * NBUF),
        *([pltpu.VMEM((CB,), jnp.int32)] * NBUF),
        *([pltpu.VMEM((CB,), jnp.float32)] * NBUF),
        *([pltpu.VMEM((CB,), jnp.float32)] * NBUF),
        pltpu.VMEM((Z,), jnp.float32),
        *([pltpu.SemaphoreType.DMA] * NBUF),
        *([pltpu.SemaphoreType.DMA] * NBUF),
        *([pltpu.SemaphoreType.DMA] * NBUF),
    ],
)
def _spmv_sc(u_hbm, rows_hbm, cols_hbm, vals_hbm, out_hbm,
             u_s, acc_s,
             rows_v0, rows_v1, rows_v2, rows_v3,
             cols_v0, cols_v1, cols_v2, cols_v3,
             vals_v0, vals_v1, vals_v2, vals_v3,
             g_v0, g_v1, g_v2, g_v3, z_v,
             si0, si1, si2, si3, sg0, sg1, sg2, sg3, ss0, ss1, ss2, ss3):
    rows_b = (rows_v0, rows_v1, rows_v2, rows_v3)
    cols_b = (cols_v0, cols_v1, cols_v2, cols_v3)
    vals_b = (vals_v0, vals_v1, vals_v2, vals_v3)
    g_b = (g_v0, g_v1, g_v2, g_v3)
    sem_in = (si0, si1, si2, si3)
    sem_g = (sg0, sg1, sg2, sg3)
    sem_sc = (ss0, ss1, ss2, ss3)

    cid = lax.axis_index("c")
    sid = lax.axis_index("s")
    wid = sid * 2 + cid

    def start_in(gidx, b):
        @pl.when(gidx < FULL_CHUNKS)
        def _():
            off = gidx * C
            pltpu.async_copy(rows_hbm.at[pl.ds(off, C)],
                             rows_b[b].at[pl.ds(0, C)], sem_in[b])
            pltpu.async_copy(cols_hbm.at[pl.ds(off, C)],
                             cols_b[b].at[pl.ds(0, C)], sem_in[b])
            pltpu.async_copy(vals_hbm.at[pl.ds(off, C)],
                             vals_b[b].at[pl.ds(0, C)], sem_in[b])

        @pl.when(gidx == FULL_CHUNKS)
        def _():
            pltpu.async_copy(rows_hbm.at[pl.ds(SP_OFF, SP_CNT)],
                             rows_b[b].at[pl.ds(0, SP_CNT)], sem_in[b])
            pltpu.async_copy(cols_hbm.at[pl.ds(SP_OFF, SP_CNT)],
                             cols_b[b].at[pl.ds(0, SP_CNT)], sem_in[b])
            pltpu.async_copy(vals_hbm.at[pl.ds(SP_OFF, SP_CNT)],
                             vals_b[b].at[pl.ds(0, SP_CNT)], sem_in[b])

    def wait_in(gidx, b):
        @pl.when(gidx < FULL_CHUNKS)
        def _():
            pltpu.make_async_copy(rows_hbm.at[pl.ds(0, C)],
                                  rows_b[b].at[pl.ds(0, C)], sem_in[b]).wait()
            pltpu.make_async_copy(cols_hbm.at[pl.ds(0, C)],
                                  cols_b[b].at[pl.ds(0, C)], sem_in[b]).wait()
            pltpu.make_async_copy(vals_hbm.at[pl.ds(0, C)],
                                  vals_b[b].at[pl.ds(0, C)], sem_in[b]).wait()

        @pl.when(gidx == FULL_CHUNKS)
        def _():
            pltpu.make_async_copy(rows_hbm.at[pl.ds(0, SP_CNT)],
                                  rows_b[b].at[pl.ds(0, SP_CNT)], sem_in[b]).wait()
            pltpu.make_async_copy(cols_hbm.at[pl.ds(0, SP_CNT)],
                                  cols_b[b].at[pl.ds(0, SP_CNT)], sem_in[b]).wait()
            pltpu.make_async_copy(vals_hbm.at[pl.ds(0, SP_CNT)],
                                  vals_b[b].at[pl.ds(0, SP_CNT)], sem_in[b]).wait()

    H = CB // 2

    def start_gather(b):
        pltpu.async_copy(u_s.at[cols_b[b].at[pl.ds(0, H)]],
                         g_b[b].at[pl.ds(0, H)], sem_g[b])
        pltpu.async_copy(u_s.at[cols_b[b].at[pl.ds(H, H)]],
                         g_b[b].at[pl.ds(H, H)], sem_g[b])

    def wait_gather(b):
        pltpu.make_async_copy(u_s.at[cols_b[b].at[pl.ds(0, H)]],
                              g_b[b].at[pl.ds(0, H)], sem_g[b]).wait()
        pltpu.make_async_copy(u_s.at[cols_b[b].at[pl.ds(H, H)]],
                              g_b[b].at[pl.ds(H, H)], sem_g[b]).wait()

    def start_sc(b):
        pltpu.async_copy(g_b[b], acc_s.at[rows_b[b]], sem_sc[b], add=True)

    def wait_sc(b):
        pltpu.make_async_copy(g_b[b], acc_s.at[rows_b[b]], sem_sc[b]).wait()

    def gi(t):
        return t * NW + wid

    start_in(gi(0), 0)
    start_in(gi(1), 1)
    pltpu.async_copy(u_hbm.at[pl.ds(sid * U_SL, U_SL)],
                     u_s.at[pl.ds(sid * U_SL, U_SL)], sem_g[0])

    def zset(i, _):
        z_v[pl.ds(i * 16, 16)] = jnp.zeros((16,), jnp.float32)
        return 0
    lax.fori_loop(0, Z // 16, zset, 0)

    def zcpy(j, _):
        pltpu.sync_copy(z_v, acc_s.at[pl.ds(sid * U_SL + j * Z, Z)])
        return 0
    lax.fori_loop(0, U_SL // Z, zcpy, 0)

    widv = jnp.full((16,), wid, jnp.int32)
    for b in range(NBUF):
        for i in range(C // 16, CB // 16):
            s = pl.ds(i * 16, 16)
            rows_b[b][s] = widv
            cols_b[b][s] = widv
            vals_b[b][s] = jnp.zeros((16,), jnp.float32)

    pltpu.make_async_copy(u_hbm.at[pl.ds(sid * U_SL, U_SL)],
                          u_s.at[pl.ds(sid * U_SL, U_SL)], sem_g[0]).wait()
    plsc.subcore_barrier()

    wait_in(gi(0), 0)
    start_gather(0)

    def consume(t, k):
        b = k
        wait_gather(b)
        gb, vb = g_b[b], vals_b[b]

        @pl.when(gi(t) == FULL_CHUNKS)
        def _():
            keep = jax.lax.iota(jnp.int32, 16) < SP_R
            sb = pl.ds(SP_G * 16, 16)
            vb[sb] = jnp.where(keep, vb[sb], 0.0)

            def zdup(i, _):
                vb[pl.ds(i * 16, 16)] = jnp.zeros((16,), jnp.float32)
                return 0
            lax.fori_loop(SP_G + 1, C // 16, zdup, 0)

        def mul(i, _):
            for r in range(MU):
                s = pl.ds(i * (16 * MU) + r * 16, 16)
                gb[s] = gb[s] * vb[s]
            return 0
        lax.fori_loop(0, CB // (16 * MU), mul, 0)

        start_sc(b)

    def body4(j, _):
        for k in range(NBUF):
            t = j * NBUF + k

            @pl.when(t >= 2)
            def _():
                wait_sc((k - 2) % NBUF)

            start_in(gi(t + 2), (k + 2) % NBUF)
            wait_in(gi(t + 1), (k + 1) % NBUF)
            start_gather((k + 1) % NBUF)
            consume(t, k)
        return 0
    lax.fori_loop(0, (T - 2) // NBUF, body4, 0)

    kA, kB = (T - 2) % NBUF, (T - 1) % NBUF
    wait_sc((kA - 2) % NBUF)
    wait_in(gi(T - 1), kB)
    start_gather(kB)
    consume(T - 2, kA)
    wait_sc((kB - 2) % NBUF)
    consume(T - 1, kB)
    wait_sc(kA)
    wait_sc(kB)

    plsc.subcore_barrier()
    pltpu.sync_copy(acc_s.at[pl.ds(sid * U_SL, U_SL)],
                    out_hbm.at[pl.ds(cid * N_PAD + sid * U_SL, U_SL)])


def _add_body(p_ref, o_ref):
    o_ref[...] = p_ref[pl.ds(0, N_PAD)] + p_ref[pl.ds(N_PAD, N_PAD)]


def kernel(u, A_rows, A_cols, A_vals):
    u_p = jnp.concatenate(
        [u, jnp.zeros((N_PAD - N_DOF, 1), jnp.float32)]).reshape(N_PAD)
    partials = _spmv_sc(u_p, A_rows.astype(jnp.int32),
                        A_cols.astype(jnp.int32), A_vals)

    summed = pl.pallas_call(
        _add_body,
        out_shape=jax.ShapeDtypeStruct((N_PAD,), jnp.float32),
    )(partials)
    return summed[:N_DOF, None]


# speedup vs baseline: 1.2527x; 1.0084x over previous
import functools

import jax
import jax.numpy as jnp
from jax import lax
from jax.experimental import pallas as pl
from jax.experimental.pallas import tpu as pltpu
from jax.experimental.pallas import tpu_sc as plsc

N_DOF = 261121
NNZ = 1827847
NW = 32
C = 5728
CB = 5760
T = 10
NBUF = 4
MU = 12
N_PAD = 262144
U_SL = N_PAD // 16
Z = 4096

S_CHUNKS = NW * T
FULL_CHUNKS = S_CHUNKS - 1
SP_OFF = FULL_CHUNKS * C
SP_CNT = NNZ - SP_OFF
SP_G = SP_CNT // 16
SP_R = SP_CNT % 16

_mesh = plsc.VectorSubcoreMesh(core_axis_name="c", subcore_axis_name="s")


@functools.partial(
    pl.kernel,
    out_type=jax.ShapeDtypeStruct((2 * N_PAD,), jnp.float32),
    mesh=_mesh,
    scratch_types=[
        pltpu.VMEM_SHARED((N_PAD,), jnp.float32),
        pltpu.VMEM_SHARED((N_PAD,), jnp.float32),
        *([pltpu.VMEM((CB,), jnp.int32)] * NBUF),
        *([pltpu.VMEM((CB,), jnp.int32)] * NBUF),
        *([pltpu.VMEM((CB,), jnp.float32)] * NBUF),
        *([pltpu.VMEM((CB,), jnp.float32)] * NBUF),
        pltpu.VMEM((Z,), jnp.float32),
        *([pltpu.SemaphoreType.DMA] * NBUF),
        *([pltpu.SemaphoreType.DMA] * NBUF),
        *([pltpu.SemaphoreType.DMA] * NBUF),
        *([pltpu.SemaphoreType.DMA] * NBUF),
    ],
)
def _spmv_sc(u_hbm, rows_hbm, cols_hbm, vals_hbm, out_hbm,
             u_s, acc_s,
             rows_v0, rows_v1, rows_v2, rows_v3,
             cols_v0, cols_v1, cols_v2, cols_v3,
             vals_v0, vals_v1, vals_v2, vals_v3,
             g_v0, g_v1, g_v2, g_v3, z_v,
             si0, si1, si2, si3, sg0, sg1, sg2, sg3,
             sh0, sh1, sh2, sh3, ss0, ss1, ss2, ss3):
    rows_b = (rows_v0, rows_v1, rows_v2, rows_v3)
    cols_b = (cols_v0, cols_v1, cols_v2, cols_v3)
    vals_b = (vals_v0, vals_v1, vals_v2, vals_v3)
    g_b = (g_v0, g_v1, g_v2, g_v3)
    sem_in = (si0, si1, si2, si3)
    sem_g = (sg0, sg1, sg2, sg3)
    sem_h = (sh0, sh1, sh2, sh3)
    sem_sc = (ss0, ss1, ss2, ss3)

    cid = lax.axis_index("c")
    sid = lax.axis_index("s")
    wid = sid * 2 + cid

    def start_in(gidx, b):
        @pl.when(gidx < FULL_CHUNKS)
        def _():
            off = gidx * C
            pltpu.async_copy(rows_hbm.at[pl.ds(off, C)],
                             rows_b[b].at[pl.ds(0, C)], sem_in[b])
            pltpu.async_copy(cols_hbm.at[pl.ds(off, C)],
                             cols_b[b].at[pl.ds(0, C)], sem_in[b])
            pltpu.async_copy(vals_hbm.at[pl.ds(off, C)],
                             vals_b[b].at[pl.ds(0, C)], sem_in[b])

        @pl.when(gidx == FULL_CHUNKS)
        def _():
            pltpu.async_copy(rows_hbm.at[pl.ds(SP_OFF, SP_CNT)],
                             rows_b[b].at[pl.ds(0, SP_CNT)], sem_in[b])
            pltpu.async_copy(cols_hbm.at[pl.ds(SP_OFF, SP_CNT)],
                             cols_b[b].at[pl.ds(0, SP_CNT)], sem_in[b])
            pltpu.async_copy(vals_hbm.at[pl.ds(SP_OFF, SP_CNT)],
                             vals_b[b].at[pl.ds(0, SP_CNT)], sem_in[b])

    def wait_in(gidx, b):
        @pl.when(gidx < FULL_CHUNKS)
        def _():
            pltpu.make_async_copy(rows_hbm.at[pl.ds(0, C)],
                                  rows_b[b].at[pl.ds(0, C)], sem_in[b]).wait()
            pltpu.make_async_copy(cols_hbm.at[pl.ds(0, C)],
                                  cols_b[b].at[pl.ds(0, C)], sem_in[b]).wait()
            pltpu.make_async_copy(vals_hbm.at[pl.ds(0, C)],
                                  vals_b[b].at[pl.ds(0, C)], sem_in[b]).wait()

        @pl.when(gidx == FULL_CHUNKS)
        def _():
            pltpu.make_async_copy(rows_hbm.at[pl.ds(0, SP_CNT)],
                                  rows_b[b].at[pl.ds(0, SP_CNT)], sem_in[b]).wait()
            pltpu.make_async_copy(cols_hbm.at[pl.ds(0, SP_CNT)],
                                  cols_b[b].at[pl.ds(0, SP_CNT)], sem_in[b]).wait()
            pltpu.make_async_copy(vals_hbm.at[pl.ds(0, SP_CNT)],
                                  vals_b[b].at[pl.ds(0, SP_CNT)], sem_in[b]).wait()

    H = CB // 2

    def start_gather(b):
        pltpu.async_copy(u_s.at[cols_b[b].at[pl.ds(0, H)]],
                         g_b[b].at[pl.ds(0, H)], sem_g[b])
        pltpu.async_copy(u_s.at[cols_b[b].at[pl.ds(H, H)]],
                         g_b[b].at[pl.ds(H, H)], sem_h[b])

    def wait_gather_half(b, h):
        if h == 0:
            pltpu.make_async_copy(u_s.at[cols_b[b].at[pl.ds(0, H)]],
                                  g_b[b].at[pl.ds(0, H)], sem_g[b]).wait()
        else:
            pltpu.make_async_copy(u_s.at[cols_b[b].at[pl.ds(H, H)]],
                                  g_b[b].at[pl.ds(H, H)], sem_h[b]).wait()

    def start_sc(b):
        pltpu.async_copy(g_b[b], acc_s.at[rows_b[b]], sem_sc[b], add=True)

    def wait_sc(b):
        pltpu.make_async_copy(g_b[b], acc_s.at[rows_b[b]], sem_sc[b]).wait()

    def gi(t):
        return t * NW + wid

    start_in(gi(0), 0)
    start_in(gi(1), 1)
    pltpu.async_copy(u_hbm.at[pl.ds(sid * U_SL, U_SL)],
                     u_s.at[pl.ds(sid * U_SL, U_SL)], sem_g[0])

    widv = jnp.full((16,), wid, jnp.int32)
    for b in range(NBUF):
        for i in range(C // 16, CB // 16):
            s = pl.ds(i * 16, 16)
            rows_b[b][s] = widv
            cols_b[b][s] = widv
            vals_b[b][s] = jnp.zeros((16,), jnp.float32)

    pltpu.make_async_copy(u_hbm.at[pl.ds(sid * U_SL, U_SL)],
                          u_s.at[pl.ds(sid * U_SL, U_SL)], sem_g[0]).wait()
    plsc.subcore_barrier()

    wait_in(gi(0), 0)
    start_gather(0)

    def zset(i, _):
        z_v[pl.ds(i * 16, 16)] = jnp.zeros((16,), jnp.float32)
        return 0
    lax.fori_loop(0, Z // 16, zset, 0)

    def zcpy(j, _):
        pltpu.sync_copy(z_v, acc_s.at[pl.ds(sid * U_SL + j * Z, Z)])
        return 0
    lax.fori_loop(0, U_SL // Z, zcpy, 0)

    plsc.subcore_barrier()

    def consume(t, k):
        b = k
        gb, vb = g_b[b], vals_b[b]

        @pl.when(gi(t) == FULL_CHUNKS)
        def _():
            keep = jax.lax.iota(jnp.int32, 16) < SP_R
            sb = pl.ds(SP_G * 16, 16)
            vb[sb] = jnp.where(keep, vb[sb], 0.0)

            def zdup(i, _):
                vb[pl.ds(i * 16, 16)] = jnp.zeros((16,), jnp.float32)
                return 0
            lax.fori_loop(SP_G + 1, C // 16, zdup, 0)

        def mul(i, _):
            for r in range(MU):
                s = pl.ds(i * (16 * MU) + r * 16, 16)
                gb[s] = gb[s] * vb[s]
            return 0
        wait_gather_half(b, 0)
        lax.fori_loop(0, H // (16 * MU), mul, 0)
        wait_gather_half(b, 1)
        lax.fori_loop(H // (16 * MU), CB // (16 * MU), mul, 0)

        start_sc(b)

    def body4(j, _):
        for k in range(NBUF):
            t = j * NBUF + k

            @pl.when(t >= 2)
            def _():
                wait_sc((k - 2) % NBUF)

            start_in(gi(t + 2), (k + 2) % NBUF)
            wait_in(gi(t + 1), (k + 1) % NBUF)
            start_gather((k + 1) % NBUF)
            consume(t, k)
        return 0
    lax.fori_loop(0, (T - 2) // NBUF, body4, 0)

    kA, kB = (T - 2) % NBUF, (T - 1) % NBUF
    wait_sc((kA - 2) % NBUF)
    wait_in(gi(T - 1), kB)
    start_gather(kB)
    consume(T - 2, kA)
    wait_sc((kB - 2) % NBUF)
    consume(T - 1, kB)
    wait_sc(kA)
    wait_sc(kB)

    plsc.subcore_barrier()
    pltpu.sync_copy(acc_s.at[pl.ds(sid * U_SL, U_SL)],
                    out_hbm.at[pl.ds(cid * N_PAD + sid * U_SL, U_SL)])


def _add_body(p_ref, o_ref):
    o_ref[...] = p_ref[pl.ds(0, N_PAD)] + p_ref[pl.ds(N_PAD, N_PAD)]


def kernel(u, A_rows, A_cols, A_vals):
    u_p = jnp.concatenate(
        [u, jnp.zeros((N_PAD - N_DOF, 1), jnp.float32)]).reshape(N_PAD)
    partials = _spmv_sc(u_p, A_rows.astype(jnp.int32),
                        A_cols.astype(jnp.int32), A_vals)

    summed = pl.pallas_call(
        _add_body,
        out_shape=jax.ShapeDtypeStruct((N_PAD,), jnp.float32),
    )(partials)
    return summed[:N_DOF, None]
